# baseline (speedup 1.0000x reference)
import jax
import jax.numpy as jnp
from jax.experimental import pallas as pl
from jax.experimental.pallas import tpu as pltpu


def _make_body(num_rows):
    def _dma_body(idx_ref, src, fast, slow, row_sem, fast_sem):
        pltpu.make_async_copy(src, fast, fast_sem).start()
        for k in range(num_rows):
            pltpu.make_async_copy(src.at[idx_ref[k]], slow.at[k], row_sem).start()
        for k in range(num_rows):
            pltpu.make_async_copy(src.at[idx_ref[k]], slow.at[k], row_sem).wait()
        pltpu.make_async_copy(src, fast, fast_sem).wait()

    return _dma_body


def kernel(frames):
    C, T, H, W = frames.shape
    alpha = 4
    n = T // alpha
    idx = jnp.linspace(0.0, float(T - 1), n).astype(jnp.int32)
    row_idx = (jnp.arange(C, dtype=jnp.int32)[:, None] * T + idx[None, :]).reshape(-1)

    flat = frames.reshape(C * T, H, W)
    hbm = pl.BlockSpec(memory_space=pltpu.MemorySpace.HBM)
    fast_flat, slow_flat = pl.pallas_call(
        _make_body(C * n),
        grid_spec=pltpu.PrefetchScalarGridSpec(
            num_scalar_prefetch=1,
            grid=(1,),
            in_specs=[hbm],
            out_specs=[hbm, hbm],
            scratch_shapes=[pltpu.SemaphoreType.DMA, pltpu.SemaphoreType.DMA],
        ),
        out_shape=[
            jax.ShapeDtypeStruct((C * T, H, W), jnp.float32),
            jax.ShapeDtypeStruct((C * n, H, W), jnp.float32),
        ],
    )(row_idx, flat)
    return (slow_flat.reshape(C, n, H, W), fast_flat.reshape(C, T, H, W))


# speedup vs baseline: 52.3682x; 52.3682x over previous
import jax
import jax.numpy as jnp
import numpy as np
from jax.experimental import pallas as pl
from jax.experimental.pallas import tpu as pltpu

_G = 8
_NBUF = 6
_LAG = 3


def _make_body(nrows, slow_map):
    nchunks = nrows // _G

    def body(src, fast, slow, bufs, in_sems, out_sems):
        def in_copy(g):
            b = g % _NBUF
            return pltpu.make_async_copy(
                src.at[pl.ds(g * _G, _G)], bufs.at[b], in_sems.at[b])

        def out_copies(g):
            b = g % _NBUF
            cps = [pltpu.make_async_copy(
                bufs.at[b], fast.at[pl.ds(g * _G, _G)], out_sems.at[b])]
            for k, r in slow_map[g]:
                cps.append(pltpu.make_async_copy(
                    bufs.at[b].at[r], slow.at[k], out_sems.at[b]))
            return cps

        for g in range(min(_NBUF, nchunks)):
            in_copy(g).start()
        for g in range(nchunks):
            in_copy(g).wait()
            for cp in out_copies(g):
                cp.start()
            gl = g - _LAG
            if gl >= 0:
                for cp in out_copies(gl):
                    cp.wait()
                if gl + _NBUF < nchunks:
                    in_copy(gl + _NBUF).start()
        for g in range(max(0, nchunks - _LAG), nchunks):
            for cp in out_copies(g):
                cp.wait()

    return body


def kernel(frames):
    C, T, H, W = frames.shape
    alpha = 4
    n = T // alpha
    with jax.ensure_compile_time_eval():
        idx = np.asarray(jnp.linspace(0.0, float(T - 1), n).astype(jnp.int32))

    nrows = C * T
    slow_map = {g: [] for g in range(nrows // _G)}
    for c in range(C):
        for j, t in enumerate(idx.tolist()):
            r = c * T + t
            slow_map[r // _G].append((c * n + j, r % _G))

    flat = frames.reshape(nrows, H, W)
    hbm = pl.BlockSpec(memory_space=pltpu.MemorySpace.HBM)
    fast_flat, slow_flat = pl.pallas_call(
        _make_body(nrows, slow_map),
        in_specs=[hbm],
        out_specs=[hbm, hbm],
        out_shape=[
            jax.ShapeDtypeStruct((nrows, H, W), jnp.float32),
            jax.ShapeDtypeStruct((C * n, H, W), jnp.float32),
        ],
        scratch_shapes=[
            pltpu.VMEM((_NBUF, _G, H, W), jnp.float32),
            pltpu.SemaphoreType.DMA((_NBUF,)),
            pltpu.SemaphoreType.DMA((_NBUF,)),
        ],
    )(flat)
    return (slow_flat.reshape(C, n, H, W), fast_flat.reshape(C, T, H, W))


# speedup vs baseline: 53.3021x; 1.0178x over previous
import jax
import jax.numpy as jnp
import numpy as np
from jax.experimental import pallas as pl
from jax.experimental.pallas import tpu as pltpu

_G = 16
_NBUF = 4
_LAG = 2


def _make_body(nrows, slow_map):
    nchunks = nrows // _G

    def body(src, fast, slow, bufs, in_sems, out_sems):
        def in_copy(g):
            b = g % _NBUF
            return pltpu.make_async_copy(
                src.at[pl.ds(g * _G, _G)], bufs.at[b], in_sems.at[b])

        def out_copies(g):
            b = g % _NBUF
            cps = [pltpu.make_async_copy(
                bufs.at[b], fast.at[pl.ds(g * _G, _G)], out_sems.at[b])]
            for k, r in slow_map[g]:
                cps.append(pltpu.make_async_copy(
                    bufs.at[b].at[r], slow.at[k], out_sems.at[b]))
            return cps

        for g in range(min(_NBUF, nchunks)):
            in_copy(g).start()
        for g in range(nchunks):
            in_copy(g).wait()
            for cp in out_copies(g):
                cp.start()
            gl = g - _LAG
            if gl >= 0:
                for cp in out_copies(gl):
                    cp.wait()
                if gl + _NBUF < nchunks:
                    in_copy(gl + _NBUF).start()
        for g in range(max(0, nchunks - _LAG), nchunks):
            for cp in out_copies(g):
                cp.wait()

    return body


def kernel(frames):
    C, T, H, W = frames.shape
    alpha = 4
    n = T // alpha
    with jax.ensure_compile_time_eval():
        idx = np.asarray(jnp.linspace(0.0, float(T - 1), n).astype(jnp.int32))

    nrows = C * T
    slow_map = {g: [] for g in range(nrows // _G)}
    for c in range(C):
        for j, t in enumerate(idx.tolist()):
            r = c * T + t
            slow_map[r // _G].append((c * n + j, r % _G))

    flat = frames.reshape(nrows, H, W)
    hbm = pl.BlockSpec(memory_space=pltpu.MemorySpace.HBM)
    fast_flat, slow_flat = pl.pallas_call(
        _make_body(nrows, slow_map),
        in_specs=[hbm],
        out_specs=[hbm, hbm],
        out_shape=[
            jax.ShapeDtypeStruct((nrows, H, W), jnp.float32),
            jax.ShapeDtypeStruct((C * n, H, W), jnp.float32),
        ],
        scratch_shapes=[
            pltpu.VMEM((_NBUF, _G, H, W), jnp.float32),
            pltpu.SemaphoreType.DMA((_NBUF,)),
            pltpu.SemaphoreType.DMA((_NBUF,)),
        ],
    )(flat)
    return (slow_flat.reshape(C, n, H, W), fast_flat.reshape(C, T, H, W))


# speedup vs baseline: 54.6614x; 1.0255x over previous
import jax
import jax.numpy as jnp
import numpy as np
from jax.experimental import pallas as pl
from jax.experimental.pallas import tpu as pltpu

_G = 32
_NBUF = 4
_LAG = 2


def _make_body(nrows, slow_map):
    nchunks = nrows // _G

    def body(src, fast, slow, bufs, in_sems, out_sems):
        def in_copy(g):
            b = g % _NBUF
            return pltpu.make_async_copy(
                src.at[pl.ds(g * _G, _G)], bufs.at[b], in_sems.at[b])

        def out_copies(g):
            b = g % _NBUF
            cps = [pltpu.make_async_copy(
                bufs.at[b], fast.at[pl.ds(g * _G, _G)], out_sems.at[b])]
            for k, r in slow_map[g]:
                cps.append(pltpu.make_async_copy(
                    bufs.at[b].at[r], slow.at[k], out_sems.at[b]))
            return cps

        for g in range(min(_NBUF, nchunks)):
            in_copy(g).start()
        for g in range(nchunks):
            in_copy(g).wait()
            for cp in out_copies(g):
                cp.start()
            gl = g - _LAG
            if gl >= 0:
                for cp in out_copies(gl):
                    cp.wait()
                if gl + _NBUF < nchunks:
                    in_copy(gl + _NBUF).start()
        for g in range(max(0, nchunks - _LAG), nchunks):
            for cp in out_copies(g):
                cp.wait()

    return body


def kernel(frames):
    C, T, H, W = frames.shape
    alpha = 4
    n = T // alpha
    with jax.ensure_compile_time_eval():
        idx = np.asarray(jnp.linspace(0.0, float(T - 1), n).astype(jnp.int32))

    nrows = C * T
    slow_map = {g: [] for g in range(nrows // _G)}
    for c in range(C):
        for j, t in enumerate(idx.tolist()):
            r = c * T + t
            slow_map[r // _G].append((c * n + j, r % _G))

    flat = frames.reshape(nrows, H, W)
    hbm = pl.BlockSpec(memory_space=pltpu.MemorySpace.HBM)
    fast_flat, slow_flat = pl.pallas_call(
        _make_body(nrows, slow_map),
        in_specs=[hbm],
        out_specs=[hbm, hbm],
        out_shape=[
            jax.ShapeDtypeStruct((nrows, H, W), jnp.float32),
            jax.ShapeDtypeStruct((C * n, H, W), jnp.float32),
        ],
        scratch_shapes=[
            pltpu.VMEM((_NBUF, _G, H, W), jnp.float32),
            pltpu.SemaphoreType.DMA((_NBUF,)),
            pltpu.SemaphoreType.DMA((_NBUF,)),
        ],
    )(flat)
    return (slow_flat.reshape(C, n, H, W), fast_flat.reshape(C, T, H, W))
